# baseline (speedup 1.0000x reference)
import functools

import jax
import jax.numpy as jnp
from jax import lax
from jax.experimental import pallas as pl
from jax.experimental.pallas import tpu as pltpu
from jax.experimental.pallas import tpu_sc as plsc

QL = 1000
QE = 32
CH = 26
B = 4096
L = 20

NW = 32
PAIRS = CH * L
BCHUNK = 512
NCHUNK = B // BCHUNK
QT = QE // 8
BT = B // 128
BTC = BCHUNK // 128


def _sc_body(x_hbm, emb_hbm, out_hbm, table_v, x_v, out_v, qv_v):
    wid = lax.axis_index("s") * 2 + lax.axis_index("c")
    start = (wid * PAIRS) // NW
    end = ((wid + 1) * PAIRS) // NW

    def pair_body(p, cur_c):
        c = p // L
        l = p - c * L

        @pl.when(c != cur_c)
        def _():
            pltpu.sync_copy(emb_hbm.at[c], table_v)

        def chunk_body(k, carry):
            pltpu.sync_copy(x_hbm.at[c, l, pl.ds(k * BCHUNK, BCHUNK)], x_v)

            def jo_body(jo, carry2):
                for ji in range(8):
                    idx16 = x_v[pl.ds(jo * 128 + ji * 16, 16)]
                    for q in range(QE):
                        val = plsc.load_gather(
                            table_v, [qv_v[pl.ds(q * 16, 16)], idx16])
                        out_v[q // 8, jo, q % 8, pl.ds(ji * 16, 16)] = val
                return carry2

            lax.fori_loop(0, BTC, jo_body, 0)
            pltpu.sync_copy(out_v, out_hbm.at[c, l, :, pl.ds(k * BTC, BTC)])
            return carry

        lax.fori_loop(0, NCHUNK, chunk_body, 0)
        return c

    lax.fori_loop(start, end, pair_body, jnp.int32(-1))


@functools.partial(
    pl.kernel,
    out_type=jax.ShapeDtypeStruct((CH, L, QT, BT, 8, 128), jnp.float32),
    mesh=plsc.VectorSubcoreMesh(core_axis_name="c", subcore_axis_name="s"),
    scratch_types=[
        pltpu.VMEM((QE, QL), jnp.float32),
        pltpu.VMEM((BCHUNK,), jnp.int32),
        pltpu.VMEM((QT, BTC, 8, 128), jnp.float32),
    ],
    compiler_params=pltpu.CompilerParams(
        use_tc_tiling_on_sc=False, needs_layout_passes=False),
)
def _sc_gather(x_hbm, emb_hbm, qv_hbm, out_hbm, table_v, x_v, out_v):
    qv_shape = (QE * 16,)
    del qv_shape

    def inner(qv_v):
        pltpu.sync_copy(qv_hbm, qv_v)
        _sc_body(x_hbm, emb_hbm, out_hbm, table_v, x_v, out_v, qv_v)

    pl.run_scoped(inner, pltpu.VMEM((QE * 16,), jnp.int32))


def kernel(x, emb):
    x_t = jnp.transpose(x, (1, 2, 0)).astype(jnp.int32)
    emb_t = jnp.transpose(emb, (0, 2, 1))
    qv = jnp.repeat(jnp.arange(QE, dtype=jnp.int32), 16)
    out6 = _sc_gather(x_t, emb_t, qv)
    out = jnp.transpose(out6, (3, 5, 0, 1, 2, 4)).reshape(B, CH, L, QE)
    return out


# speedup vs baseline: 3.9632x; 3.9632x over previous
import functools

import jax
import jax.numpy as jnp
from jax import lax
from jax.experimental import pallas as pl
from jax.experimental.pallas import tpu as pltpu
from jax.experimental.pallas import tpu_sc as plsc

QL = 1000
QE = 32
CH = 26
B = 4096
L = 20

NW = 32
PAIRS = CH * L
BCHUNK = 512
NCHUNK = B // BCHUNK
QT = QE // 8
BT = B // 128
BTC = BCHUNK // 128


def _sc_body(x_hbm, emb_hbm, out_hbm, table_v, x_v, out_v):
    wid = lax.axis_index("s") * 2 + lax.axis_index("c")
    start = (wid * PAIRS) // NW
    end = ((wid + 1) * PAIRS) // NW

    def pair_body(p, cur_c):
        c = p // L
        l = p - c * L

        @pl.when(c != cur_c)
        def _():
            pltpu.sync_copy(emb_hbm.at[c], table_v)

        def chunk_body(k, carry):
            pltpu.sync_copy(x_hbm.at[c, l, pl.ds(k * BCHUNK, BCHUNK)], x_v)

            @plsc.parallel_loop(0, BCHUNK // 16, unroll=4)
            def _(j):
                idx16 = x_v[pl.ds(j * 16, 16)]
                jt = j // 8
                jr = (j % 8) * 16
                for q in range(QE):
                    qv = jnp.full((16,), q, jnp.int32)
                    val = plsc.load_gather(table_v, [qv, idx16])
                    out_v[q // 8, jt, q % 8, pl.ds(jr, 16)] = val

            pltpu.sync_copy(out_v, out_hbm.at[c, l, :, pl.ds(k * BTC, BTC)])
            return carry

        lax.fori_loop(0, NCHUNK, chunk_body, 0)
        return c

    lax.fori_loop(start, end, pair_body, jnp.int32(-1))


@functools.partial(
    pl.kernel,
    out_type=jax.ShapeDtypeStruct((CH, L, QT, BT, 8, 128), jnp.float32),
    mesh=plsc.VectorSubcoreMesh(core_axis_name="c", subcore_axis_name="s"),
    scratch_types=[
        pltpu.VMEM((QE, QL), jnp.float32),
        pltpu.VMEM((BCHUNK,), jnp.int32),
        pltpu.VMEM((QT, BTC, 8, 128), jnp.float32),
    ],
    compiler_params=pltpu.CompilerParams(
        use_tc_tiling_on_sc=False, needs_layout_passes=False),
)
def _sc_gather(x_hbm, emb_hbm, out_hbm, table_v, x_v, out_v):
    _sc_body(x_hbm, emb_hbm, out_hbm, table_v, x_v, out_v)


def kernel(x, emb):
    x_t = jnp.transpose(x, (1, 2, 0)).astype(jnp.int32)
    emb_t = jnp.transpose(emb, (0, 2, 1))
    out6 = _sc_gather(x_t, emb_t)
    out = jnp.transpose(out6, (3, 5, 0, 1, 2, 4)).reshape(B, CH, L, QE)
    return out


# speedup vs baseline: 6.3257x; 1.5961x over previous
import functools

import jax
import jax.numpy as jnp
from jax import lax
from jax.experimental import pallas as pl
from jax.experimental.pallas import tpu as pltpu
from jax.experimental.pallas import tpu_sc as plsc

QL = 1000
QE = 32
CH = 26
B = 4096
L = 20

NW = 32
PAIRS = CH * L
BCHUNK = 512
NCHUNK = B // BCHUNK
QT = QE // 8
BT = B // 128
BTC = BCHUNK // 128


def _sc_body(x_hbm, emb_hbm, out_hbm, table_v, x0_v, x1_v, out0_v, out1_v,
             xsem0, xsem1, wsem0, wsem1):
    wid = lax.axis_index("s") * 2 + lax.axis_index("c")
    start = (wid * PAIRS) // NW
    end = ((wid + 1) * PAIRS) // NW
    total = (end - start) * NCHUNK
    xs = (x0_v, x1_v)
    outs = (out0_v, out1_v)
    xsems = (xsem0, xsem1)
    wsems = (wsem0, wsem1)

    def _cl(t):
        p = start + t // NCHUNK
        c = p // L
        return c, p - c * L, t - (t // NCHUNK) * NCHUNK

    def _xcopy(t, b):
        c, l, k = _cl(t)
        return pltpu.make_async_copy(
            x_hbm.at[c, l, pl.ds(k * BCHUNK, BCHUNK)], xs[b], xsems[b])

    def _wcopy(t, b):
        c, l, k = _cl(t)
        return pltpu.make_async_copy(
            outs[b], out_hbm.at[c, l, :, pl.ds(k * BTC, BTC)], wsems[b])

    _xcopy(0, 0).start()

    def do_item(t, b, cur_c):
        c, l, k = _cl(t)
        _xcopy(t, b).wait()

        @pl.when(t + 1 < total)
        def _():
            _xcopy(t + 1, 1 - b).start()

        @pl.when(c != cur_c)
        def _():
            pltpu.sync_copy(emb_hbm.at[c], table_v)

        @pl.when(t >= 2)
        def _():
            _wcopy(t, b).wait()

        @plsc.parallel_loop(0, BCHUNK // 16, unroll=4)
        def _(j):
            idx16 = xs[b][pl.ds(j * 16, 16)]
            jt = j // 8
            jr = (j % 8) * 16
            for q in range(QE):
                qv = jnp.full((16,), q, jnp.int32)
                val = plsc.load_gather(table_v, [qv, idx16])
                outs[b][q // 8, jt, q % 8, pl.ds(jr, 16)] = val

        _wcopy(t, b).start()
        return c

    def pair_body(m, cur_c):
        cur_c = do_item(2 * m, 0, cur_c)
        cur_c = do_item(2 * m + 1, 1, cur_c)
        return cur_c

    lax.fori_loop(0, total // 2, pair_body, jnp.int32(-1))
    _wcopy(0, 0).wait()
    _wcopy(1, 1).wait()


@functools.partial(
    pl.kernel,
    out_type=jax.ShapeDtypeStruct((CH, L, QT, BT, 8, 128), jnp.float32),
    mesh=plsc.VectorSubcoreMesh(core_axis_name="c", subcore_axis_name="s"),
    scratch_types=[
        pltpu.VMEM((QE, QL), jnp.float32),
        pltpu.VMEM((BCHUNK,), jnp.int32),
        pltpu.VMEM((BCHUNK,), jnp.int32),
        pltpu.VMEM((QT, BTC, 8, 128), jnp.float32),
        pltpu.VMEM((QT, BTC, 8, 128), jnp.float32),
        pltpu.SemaphoreType.DMA,
        pltpu.SemaphoreType.DMA,
        pltpu.SemaphoreType.DMA,
        pltpu.SemaphoreType.DMA,
    ],
    compiler_params=pltpu.CompilerParams(
        use_tc_tiling_on_sc=False, needs_layout_passes=False),
)
def _sc_gather(x_hbm, emb_hbm, out_hbm, table_v, x0_v, x1_v, out0_v, out1_v,
               xsem0, xsem1, wsem0, wsem1):
    _sc_body(x_hbm, emb_hbm, out_hbm, table_v, x0_v, x1_v, out0_v, out1_v,
             xsem0, xsem1, wsem0, wsem1)


def kernel(x, emb):
    x_t = jnp.transpose(x, (1, 2, 0)).astype(jnp.int32)
    emb_t = jnp.transpose(emb, (0, 2, 1))
    out6 = _sc_gather(x_t, emb_t)
    out = jnp.transpose(out6, (3, 5, 0, 1, 2, 4)).reshape(B, CH, L, QE)
    return out
